# Initial kernel scaffold; baseline (speedup 1.0000x reference)
import functools

import jax
import jax.numpy as jnp
from jax import lax
from jax.experimental import pallas as pl
from jax.experimental.pallas import tpu as pltpu
from jax.experimental.pallas import tpu_sc as plsc

N = 50000
NP = 51200
E = 400000
EP = 409600
D = 128
DS = 32
NSL = D // DS
BT = 128
NB = 200
SLAB = NP // 16

_f32 = jnp.float32


def _sc_body(with_cnt, *refs):
    (gR0, gR1, gR2, gR3, gV0, gV1, gV2, gV3,
     srcR, dstR, srcV, dstV) = refs[:12]
    refs = refs[12:]
    if with_cnt:
        (sR0, sR1, sR2, sR3, sV0, sV1, sV2, sV3, cntR, cntV) = refs[:10]
        refs = refs[10:]
    else:
        (sR0, sR1, sR2, sR3, sV0, sV1, sV2, sV3) = refs[:8]
        cntR = cntV = None
        refs = refs[8:]
    (acc, cacc, src_v, dst_v, rows_v, zeros_v, zflat_v, ones_v, sem) = refs

    core = lax.axis_index("c")
    t = lax.axis_index("s")

    z16 = jnp.zeros((16,), _f32)
    o16 = jnp.ones((16,), _f32)

    def zflat_init(i, _):
        zflat_v[pl.ds(i * 16, 16)] = z16
        return 0
    lax.fori_loop(0, SLAB // 16, zflat_init, 0)

    def zeros_init(i, _):
        zeros_v[i // 2, pl.ds((i % 2) * 16, 16)] = z16
        return 0
    lax.fori_loop(0, 400 * 2, zeros_init, 0)

    def ones_init(i, _):
        ones_v[pl.ds(i * 16, 16)] = o16
        return 0
    lax.fori_loop(0, BT // 16, ones_init, 0)

    def run(gs, srcH, dstH, outs, cnt):
        pltpu.sync_copy(srcH.at[pl.ds(t * NB, NB)], src_v)
        pltpu.sync_copy(dstH.at[pl.ds(t * NB, NB)], dst_v)
        for csl in range(NSL):
            g = gs[csl]
            do_cnt = (cnt is not None) and csl == 0
            for k in range(SLAB // 400):
                pltpu.sync_copy(zeros_v, acc.at[pl.ds(t * SLAB + k * 400, 400)])
            if do_cnt:
                pltpu.sync_copy(zflat_v, cacc.at[pl.ds(t * SLAB, SLAB)])
            plsc.subcore_barrier()

            pltpu.async_copy(g.at[src_v.at[0]], rows_v.at[0], sem)

            def batch(j, _):
                pltpu.make_async_copy(
                    g.at[src_v.at[j]], rows_v.at[j % 2], sem).wait()

                @pl.when(j < NB - 1)
                def _():
                    pltpu.async_copy(
                        g.at[src_v.at[j + 1]], rows_v.at[(j + 1) % 2], sem)

                pltpu.sync_copy(rows_v.at[j % 2], acc.at[dst_v.at[j]],
                                add=True)
                if do_cnt:
                    pltpu.sync_copy(ones_v, cacc.at[dst_v.at[j]], add=True)
                return 0

            lax.fori_loop(0, NB, batch, 0)
            plsc.subcore_barrier()

            pltpu.sync_copy(acc.at[pl.ds(t * SLAB, SLAB)],
                            outs[csl].at[pl.ds(t * SLAB, SLAB)])
            if do_cnt:
                pltpu.sync_copy(cacc.at[pl.ds(t * SLAB, SLAB)],
                                cnt.at[pl.ds(t * SLAB, SLAB)])

    @pl.when(core == 0)
    def _():
        run((gR0, gR1, gR2, gR3), srcR, dstR, (sR0, sR1, sR2, sR3), cntR)

    @pl.when(core == 1)
    def _():
        run((gV0, gV1, gV2, gV3), srcV, dstV, (sV0, sV1, sV2, sV3), cntV)


def _make_sc(with_cnt):
    outs = [jax.ShapeDtypeStruct((NP, DS), _f32) for _ in range(8)]
    if with_cnt:
        outs += [jax.ShapeDtypeStruct((NP,), _f32) for _ in range(2)]
    return pl.kernel(
        functools.partial(_sc_body, with_cnt),
        out_type=outs,
        mesh=plsc.VectorSubcoreMesh(core_axis_name="c", subcore_axis_name="s"),
        scratch_types=[
            pltpu.VMEM_SHARED((NP, DS), _f32),
            pltpu.VMEM_SHARED((NP,), _f32),
            pltpu.VMEM((NB, BT), jnp.int32),
            pltpu.VMEM((NB, BT), jnp.int32),
            pltpu.VMEM((2, BT, DS), _f32),
            pltpu.VMEM((400, DS), _f32),
            pltpu.VMEM((SLAB,), _f32),
            pltpu.VMEM((BT,), _f32),
            pltpu.SemaphoreType.DMA,
        ],
        name="sc_segsum_cnt" if with_cnt else "sc_segsum",
    )


BR = 2500
GRID = N // BR


def _mm(a, b):
    return jax.lax.dot_general(a, b, (((1,), (0,)), ((), ())),
                               preferred_element_type=_f32)


def _tc_a_body(xu, xi, Wpu, bpu, Wpi, bpi, Wl0r, Wr0r, Wl0v, Wr0v,
               gu0, gu1, gu2, gu3, gi0, gi1, gi2, gi3, ru, ri):
    hu = _mm(xu[...], Wpu[...]) + bpu[...]
    hi = _mm(xi[...], Wpi[...]) + bpi[...]
    gu = _mm(hu, Wl0r[...])
    gi = _mm(hi, Wl0v[...])
    for c, ref in enumerate((gu0, gu1, gu2, gu3)):
        ref[...] = gu[:, c * DS:(c + 1) * DS]
    for c, ref in enumerate((gi0, gi1, gi2, gi3)):
        ref[...] = gi[:, c * DS:(c + 1) * DS]
    ru[...] = _mm(hu, Wr0v[...])
    ri[...] = _mm(hi, Wr0r[...])


def _ln_gelu(a, g, b):
    mu = jnp.mean(a, axis=-1, keepdims=True)
    d = a - mu
    var = jnp.mean(d * d, axis=-1, keepdims=True)
    x = d * jax.lax.rsqrt(var + 1e-5) * g + b
    return 0.5 * x * (1.0 + jax.lax.erf(x * 0.7071067811865476))


def _tc_b_body(sR0, sR1, sR2, sR3, sV0, sV1, sV2, sV3, cR, cV, ru, ri,
               bl0r, bl0v, lgu, lbu, lgi, lbi,
               Wl1r, Wr1r, Wl1v, Wr1v,
               gu0, gu1, gu2, gu3, gi0, gi1, gi2, gi3, r1u, r1i):
    sR = jnp.concatenate([sR0[...], sR1[...], sR2[...], sR3[...]], axis=1)
    sV = jnp.concatenate([sV0[...], sV1[...], sV2[...], sV3[...]], axis=1)
    invR = 1.0 / jnp.maximum(cR[...], 1.0)
    invV = 1.0 / jnp.maximum(cV[...], 1.0)
    ai = sR * invR + bl0r[...] + ri[...]
    au = sV * invV + bl0v[...] + ru[...]
    h1i = _ln_gelu(ai, lgi[...], lbi[...])
    h1u = _ln_gelu(au, lgu[...], lbu[...])
    g1u = _mm(h1u, Wl1r[...])
    g1i = _mm(h1i, Wl1v[...])
    for c, ref in enumerate((gu0, gu1, gu2, gu3)):
        ref[...] = g1u[:, c * DS:(c + 1) * DS]
    for c, ref in enumerate((gi0, gi1, gi2, gi3)):
        ref[...] = g1i[:, c * DS:(c + 1) * DS]
    r1u[...] = _mm(h1u, Wr1v[...])
    r1i[...] = _mm(h1i, Wr1r[...])


def _tc_c_body(sR0, sR1, sR2, sR3, sV0, sV1, sV2, sV3, cR, cV, r1u, r1i,
               bl1r, bl1v, ou, oi):
    sR = jnp.concatenate([sR0[...], sR1[...], sR2[...], sR3[...]], axis=1)
    sV = jnp.concatenate([sV0[...], sV1[...], sV2[...], sV3[...]], axis=1)
    invR = 1.0 / jnp.maximum(cR[...], 1.0)
    invV = 1.0 / jnp.maximum(cV[...], 1.0)
    oi[...] = sR * invR + bl1r[...] + r1i[...]
    ou[...] = sV * invV + bl1v[...] + r1u[...]


def _row_spec():
    return pl.BlockSpec((BR, D), lambda i: (i, 0))


def _slice_spec():
    return pl.BlockSpec((BR, DS), lambda i: (i, 0))


def _full_spec(shape):
    return pl.BlockSpec(shape, lambda i: tuple(0 for _ in shape))


def _cnt_spec():
    return pl.BlockSpec((BR, 1), lambda i: (i, 0))


def kernel(x_user, x_item, edge_index_rates, edge_index_rev,
           W_proj_user, b_proj_user, W_proj_item, b_proj_item,
           Wl0_rates, bl0_rates, Wr0_rates, Wl0_rev, bl0_rev, Wr0_rev,
           ln_g_user, ln_b_user, ln_g_item, ln_b_item,
           Wl1_rates, bl1_rates, Wr1_rates, Wl1_rev, bl1_rev, Wr1_rev):
    pad = EP - E
    pad_src = (jnp.arange(pad, dtype=jnp.int32) % N)
    pad_dst = N + (jnp.arange(pad, dtype=jnp.int32) % (NP - N))

    def prep(ei):
        src = jnp.concatenate([ei[0], pad_src]).reshape(EP // BT, BT)
        dst = jnp.concatenate([ei[1], pad_dst]).reshape(EP // BT, BT)
        return src, dst

    srcR, dstR = prep(edge_index_rates)
    srcV, dstV = prep(edge_index_rev)

    def r2(b):
        return b.reshape(1, D)

    tc_a = pl.pallas_call(
        _tc_a_body,
        grid=(GRID,),
        in_specs=[_row_spec(), _row_spec()] + [_full_spec((D, D)),
                  _full_spec((1, D))] * 2 + [_full_spec((D, D))] * 4,
        out_specs=[_slice_spec()] * 8 + [_row_spec()] * 2,
        out_shape=[jax.ShapeDtypeStruct((N, DS), _f32)] * 8 +
                  [jax.ShapeDtypeStruct((N, D), _f32)] * 2,
    )
    (gu0, gu1, gu2, gu3, gi0, gi1, gi2, gi3, r0u, r0i) = tc_a(
        x_user, x_item, W_proj_user, r2(b_proj_user),
        W_proj_item, r2(b_proj_item),
        Wl0_rates, Wr0_rates, Wl0_rev, Wr0_rev)

    sc0 = _make_sc(True)
    (sR0, sR1, sR2, sR3, sV0, sV1, sV2, sV3, cntR, cntV) = sc0(
        gu0, gu1, gu2, gu3, gi0, gi1, gi2, gi3, srcR, dstR, srcV, dstV)

    cR = cntR[:N].reshape(N, 1)
    cV = cntV[:N].reshape(N, 1)

    tc_b = pl.pallas_call(
        _tc_b_body,
        grid=(GRID,),
        in_specs=[_slice_spec()] * 8 + [_cnt_spec()] * 2 +
                 [_row_spec()] * 2 + [_full_spec((1, D))] * 6 +
                 [_full_spec((D, D))] * 4,
        out_specs=[_slice_spec()] * 8 + [_row_spec()] * 2,
        out_shape=[jax.ShapeDtypeStruct((N, DS), _f32)] * 8 +
                  [jax.ShapeDtypeStruct((N, D), _f32)] * 2,
    )
    (hu0, hu1, hu2, hu3, hi0, hi1, hi2, hi3, r1u, r1i) = tc_b(
        sR0, sR1, sR2, sR3, sV0, sV1, sV2, sV3, cR, cV, r0u, r0i,
        r2(bl0_rates), r2(bl0_rev),
        r2(ln_g_user), r2(ln_b_user), r2(ln_g_item), r2(ln_b_item),
        Wl1_rates, Wr1_rates, Wl1_rev, Wr1_rev)

    sc1 = _make_sc(False)
    (tR0, tR1, tR2, tR3, tV0, tV1, tV2, tV3) = sc1(
        hu0, hu1, hu2, hu3, hi0, hi1, hi2, hi3, srcR, dstR, srcV, dstV)

    tc_c = pl.pallas_call(
        _tc_c_body,
        grid=(GRID,),
        in_specs=[_slice_spec()] * 8 + [_cnt_spec()] * 2 +
                 [_row_spec()] * 2 + [_full_spec((1, D))] * 2,
        out_specs=[_row_spec()] * 2,
        out_shape=[jax.ShapeDtypeStruct((N, D), _f32)] * 2,
    )
    out_user, out_item = tc_c(
        tR0, tR1, tR2, tR3, tV0, tV1, tV2, tV3, cR, cV, r1u, r1i,
        r2(bl1_rates), r2(bl1_rev))
    return (out_user, out_item)


# speedup vs baseline: 3.1639x; 3.1639x over previous
import functools

import jax
import jax.numpy as jnp
from jax import lax
from jax.experimental import pallas as pl
from jax.experimental.pallas import tpu as pltpu
from jax.experimental.pallas import tpu_sc as plsc

N = 50000
NP = 51200
E = 400000
EP = 409600
D = 128
DS = 32
NSL = D // DS
BT = 128
NB = 200
WB = 25
SLAB = NP // 16

_f32 = jnp.float32


def _sc_body(with_cnt, *refs):
    (gR0, gR1, gR2, gR3, gV0, gV1, gV2, gV3,
     srcR, dstR, srcV, dstV) = refs[:12]
    refs = refs[12:]
    if with_cnt:
        (sR0, sR1, sR2, sR3, sV0, sV1, sV2, sV3, cntR, cntV) = refs[:10]
        refs = refs[10:]
    else:
        (sR0, sR1, sR2, sR3, sV0, sV1, sV2, sV3) = refs[:8]
        cntR = cntV = None
        refs = refs[8:]
    (acc, cacc, src_v, dst_v, rows_v, zeros_v, zflat_v, ones_v, sem) = refs

    core = lax.axis_index("c")
    t = lax.axis_index("s")

    z16 = jnp.zeros((16,), _f32)
    o16 = jnp.ones((16,), _f32)

    def zflat_init(i, _):
        zflat_v[pl.ds(i * 16, 16)] = z16
        return 0
    lax.fori_loop(0, 400 // 16, zflat_init, 0)

    def zeros_init(i, _):
        zeros_v[i // 2, pl.ds((i % 2) * 16, 16)] = z16
        return 0
    lax.fori_loop(0, 100 * 2, zeros_init, 0)

    def ones_init(i, _):
        ones_v[pl.ds(i * 16, 16)] = o16
        return 0
    lax.fori_loop(0, BT // 16, ones_init, 0)

    def run(gs, srcH, dstH, outs, cnt):
        for csl in range(NSL):
            g = gs[csl]
            do_cnt = (cnt is not None) and csl == 0
            for k in range(SLAB // 100):
                pltpu.sync_copy(zeros_v, acc.at[pl.ds(t * SLAB + k * 100, 100)])
            if do_cnt:
                for k in range(SLAB // 400):
                    pltpu.sync_copy(zflat_v,
                                    cacc.at[pl.ds(t * SLAB + k * 400, 400)])
            plsc.subcore_barrier()

            def window(w, _):
                pltpu.sync_copy(srcH.at[pl.ds(t * NB + w * WB, WB)], src_v)
                pltpu.sync_copy(dstH.at[pl.ds(t * NB + w * WB, WB)], dst_v)
                pltpu.async_copy(g.at[src_v.at[0]], rows_v.at[0], sem)

                def batch(j, _):
                    pltpu.make_async_copy(
                        g.at[src_v.at[j]], rows_v.at[j % 2], sem).wait()

                    @pl.when(j < WB - 1)
                    def _():
                        pltpu.async_copy(
                            g.at[src_v.at[j + 1]], rows_v.at[(j + 1) % 2], sem)

                    pltpu.sync_copy(rows_v.at[j % 2], acc.at[dst_v.at[j]],
                                    add=True)
                    if do_cnt:
                        pltpu.sync_copy(ones_v, cacc.at[dst_v.at[j]],
                                        add=True)
                    return 0

                lax.fori_loop(0, WB, batch, 0)
                return 0

            lax.fori_loop(0, NB // WB, window, 0)
            plsc.subcore_barrier()

            pltpu.sync_copy(acc.at[pl.ds(t * SLAB, SLAB)],
                            outs[csl].at[pl.ds(t * SLAB, SLAB)])
            if do_cnt:
                pltpu.sync_copy(cacc.at[pl.ds(t * SLAB, SLAB)],
                                cnt.at[pl.ds(t * SLAB, SLAB)])

    @pl.when(core == 0)
    def _():
        run((gR0, gR1, gR2, gR3), srcR, dstR, (sR0, sR1, sR2, sR3), cntR)

    @pl.when(core == 1)
    def _():
        run((gV0, gV1, gV2, gV3), srcV, dstV, (sV0, sV1, sV2, sV3), cntV)


def _make_sc(with_cnt):
    outs = [jax.ShapeDtypeStruct((NP, DS), _f32) for _ in range(8)]
    if with_cnt:
        outs += [jax.ShapeDtypeStruct((NP,), _f32) for _ in range(2)]
    return pl.kernel(
        functools.partial(_sc_body, with_cnt),
        out_type=outs,
        mesh=plsc.VectorSubcoreMesh(core_axis_name="c", subcore_axis_name="s"),
        scratch_types=[
            pltpu.VMEM_SHARED((NP, DS), _f32),
            pltpu.VMEM_SHARED((NP,), _f32),
            pltpu.VMEM((WB, BT), jnp.int32),
            pltpu.VMEM((WB, BT), jnp.int32),
            pltpu.VMEM((2, BT, DS), _f32),
            pltpu.VMEM((100, DS), _f32),
            pltpu.VMEM((400,), _f32),
            pltpu.VMEM((BT,), _f32),
            pltpu.SemaphoreType.DMA,
        ],
        compiler_params=pltpu.CompilerParams(use_tc_tiling_on_sc=False),
        name="sc_segsum_cnt" if with_cnt else "sc_segsum",
    )


BR = 2000
GRID = N // BR


def _mm(a, b):
    return jax.lax.dot_general(a, b, (((1,), (0,)), ((), ())),
                               preferred_element_type=_f32)


def _tc_a_body(xu, xi, Wpu, bpu, Wpi, bpi, Wl0r, Wr0r, Wl0v, Wr0v,
               gu0, gu1, gu2, gu3, gi0, gi1, gi2, gi3, ru, ri):
    hu = _mm(xu[...], Wpu[...]) + bpu[...]
    hi = _mm(xi[...], Wpi[...]) + bpi[...]
    gu = _mm(hu, Wl0r[...])
    gi = _mm(hi, Wl0v[...])
    for c, ref in enumerate((gu0, gu1, gu2, gu3)):
        ref[...] = gu[:, c * DS:(c + 1) * DS]
    for c, ref in enumerate((gi0, gi1, gi2, gi3)):
        ref[...] = gi[:, c * DS:(c + 1) * DS]
    ru[...] = _mm(hu, Wr0v[...])
    ri[...] = _mm(hi, Wr0r[...])


def _ln_gelu(a, g, b):
    mu = jnp.mean(a, axis=-1, keepdims=True)
    d = a - mu
    var = jnp.mean(d * d, axis=-1, keepdims=True)
    x = d * jax.lax.rsqrt(var + 1e-5) * g + b
    return 0.5 * x * (1.0 + jax.lax.erf(x * 0.7071067811865476))


def _tc_b_body(sR0, sR1, sR2, sR3, sV0, sV1, sV2, sV3, cR, cV, ru, ri,
               bl0r, bl0v, lgu, lbu, lgi, lbi,
               Wl1r, Wr1r, Wl1v, Wr1v,
               gu0, gu1, gu2, gu3, gi0, gi1, gi2, gi3, r1u, r1i):
    sR = jnp.concatenate([sR0[...], sR1[...], sR2[...], sR3[...]], axis=1)
    sV = jnp.concatenate([sV0[...], sV1[...], sV2[...], sV3[...]], axis=1)
    invR = 1.0 / jnp.maximum(cR[...], 1.0)
    invV = 1.0 / jnp.maximum(cV[...], 1.0)
    ai = sR * invR + bl0r[...] + ri[...]
    au = sV * invV + bl0v[...] + ru[...]
    h1i = _ln_gelu(ai, lgi[...], lbi[...])
    h1u = _ln_gelu(au, lgu[...], lbu[...])
    g1u = _mm(h1u, Wl1r[...])
    g1i = _mm(h1i, Wl1v[...])
    for c, ref in enumerate((gu0, gu1, gu2, gu3)):
        ref[...] = g1u[:, c * DS:(c + 1) * DS]
    for c, ref in enumerate((gi0, gi1, gi2, gi3)):
        ref[...] = g1i[:, c * DS:(c + 1) * DS]
    r1u[...] = _mm(h1u, Wr1v[...])
    r1i[...] = _mm(h1i, Wr1r[...])


def _tc_c_body(sR0, sR1, sR2, sR3, sV0, sV1, sV2, sV3, cR, cV, r1u, r1i,
               bl1r, bl1v, ou, oi):
    sR = jnp.concatenate([sR0[...], sR1[...], sR2[...], sR3[...]], axis=1)
    sV = jnp.concatenate([sV0[...], sV1[...], sV2[...], sV3[...]], axis=1)
    invR = 1.0 / jnp.maximum(cR[...], 1.0)
    invV = 1.0 / jnp.maximum(cV[...], 1.0)
    oi[...] = sR * invR + bl1r[...] + r1i[...]
    ou[...] = sV * invV + bl1v[...] + r1u[...]


def _row_spec():
    return pl.BlockSpec((BR, D), lambda i: (i, 0))


def _slice_spec():
    return pl.BlockSpec((BR, DS), lambda i: (i, 0))


def _full_spec(shape):
    return pl.BlockSpec(shape, lambda i: tuple(0 for _ in shape))


def _cnt_spec():
    return pl.BlockSpec((BR, 1), lambda i: (i, 0))


def kernel(x_user, x_item, edge_index_rates, edge_index_rev,
           W_proj_user, b_proj_user, W_proj_item, b_proj_item,
           Wl0_rates, bl0_rates, Wr0_rates, Wl0_rev, bl0_rev, Wr0_rev,
           ln_g_user, ln_b_user, ln_g_item, ln_b_item,
           Wl1_rates, bl1_rates, Wr1_rates, Wl1_rev, bl1_rev, Wr1_rev):
    pad = EP - E
    pad_src = (jnp.arange(pad, dtype=jnp.int32) % N)
    pad_dst = N + (jnp.arange(pad, dtype=jnp.int32) % (NP - N))

    def prep(ei):
        src = jnp.concatenate([ei[0], pad_src]).reshape(EP // BT, BT)
        dst = jnp.concatenate([ei[1], pad_dst]).reshape(EP // BT, BT)
        return src, dst

    srcR, dstR = prep(edge_index_rates)
    srcV, dstV = prep(edge_index_rev)

    def r2(b):
        return b.reshape(1, D)

    tc_a = pl.pallas_call(
        _tc_a_body,
        grid=(GRID,),
        in_specs=[_row_spec(), _row_spec()] + [_full_spec((D, D)),
                  _full_spec((1, D))] * 2 + [_full_spec((D, D))] * 4,
        out_specs=[_slice_spec()] * 8 + [_row_spec()] * 2,
        out_shape=[jax.ShapeDtypeStruct((N, DS), _f32)] * 8 +
                  [jax.ShapeDtypeStruct((N, D), _f32)] * 2,
    )
    (gu0, gu1, gu2, gu3, gi0, gi1, gi2, gi3, r0u, r0i) = tc_a(
        x_user, x_item, W_proj_user, r2(b_proj_user),
        W_proj_item, r2(b_proj_item),
        Wl0_rates, Wr0_rates, Wl0_rev, Wr0_rev)

    sc0 = _make_sc(True)
    (sR0, sR1, sR2, sR3, sV0, sV1, sV2, sV3, cntR, cntV) = sc0(
        gu0, gu1, gu2, gu3, gi0, gi1, gi2, gi3, srcR, dstR, srcV, dstV)

    cR = cntR[:N].reshape(N, 1)
    cV = cntV[:N].reshape(N, 1)

    tc_b = pl.pallas_call(
        _tc_b_body,
        grid=(GRID,),
        in_specs=[_slice_spec()] * 8 + [_cnt_spec()] * 2 +
                 [_row_spec()] * 2 + [_full_spec((1, D))] * 6 +
                 [_full_spec((D, D))] * 4,
        out_specs=[_slice_spec()] * 8 + [_row_spec()] * 2,
        out_shape=[jax.ShapeDtypeStruct((N, DS), _f32)] * 8 +
                  [jax.ShapeDtypeStruct((N, D), _f32)] * 2,
    )
    (hu0, hu1, hu2, hu3, hi0, hi1, hi2, hi3, r1u, r1i) = tc_b(
        sR0, sR1, sR2, sR3, sV0, sV1, sV2, sV3, cR, cV, r0u, r0i,
        r2(bl0_rates), r2(bl0_rev),
        r2(ln_g_user), r2(ln_b_user), r2(ln_g_item), r2(ln_b_item),
        Wl1_rates, Wr1_rates, Wl1_rev, Wr1_rev)

    sc1 = _make_sc(False)
    (tR0, tR1, tR2, tR3, tV0, tV1, tV2, tV3) = sc1(
        hu0, hu1, hu2, hu3, hi0, hi1, hi2, hi3, srcR, dstR, srcV, dstV)

    tc_c = pl.pallas_call(
        _tc_c_body,
        grid=(GRID,),
        in_specs=[_slice_spec()] * 8 + [_cnt_spec()] * 2 +
                 [_row_spec()] * 2 + [_full_spec((1, D))] * 2,
        out_specs=[_row_spec()] * 2,
        out_shape=[jax.ShapeDtypeStruct((N, D), _f32)] * 2,
    )
    out_user, out_item = tc_c(
        tR0, tR1, tR2, tR3, tV0, tV1, tV2, tV3, cR, cV, r1u, r1i,
        r2(bl1_rates), r2(bl1_rev))
    return (out_user, out_item)
